# baseline (speedup 1.0000x reference)
import functools

import jax
import jax.numpy as jnp
from jax import lax
from jax.experimental import pallas as pl
from jax.experimental.pallas import tpu as pltpu
from jax.experimental.pallas import tpu_sc as plsc

N = 10000
E = 320000
NC = 2
NS = 16
SLAB = 624
SLAB_LAST = N - SLAB * (NS - 1)
CH = 128

_MESH = dict(core_axis_name="c", subcore_axis_name="s", num_cores=NC,
             num_subcores=NS)


def _slab_copy(s, mk_src, mk_dst):
    @pl.when(s < NS - 1)
    def _():
        pltpu.sync_copy(mk_src(s * SLAB, SLAB), mk_dst(s * SLAB, SLAB))

    @pl.when(s == NS - 1)
    def _():
        pltpu.sync_copy(mk_src(SLAB * (NS - 1), SLAB_LAST),
                        mk_dst(SLAB * (NS - 1), SLAB_LAST))


def _deg_body(dst_hbm, zeros_hbm, ones_hbm, degp_hbm, ones_v, idx0_v, idx1_v,
              tidx_v, acc, sem0, sem1, tsem):
    c = lax.axis_index("c")
    s = lax.axis_index("s")
    _slab_copy(s, lambda r0, n: zeros_hbm.at[pl.ds(0, n)],
               lambda r0, n: acc.at[pl.ds(r0, n)])
    pltpu.sync_copy(ones_hbm, ones_v)
    plsc.subcore_barrier()

    ebase = (c * NS + s) * (E // (NC * NS))
    n_chunks = (E // (NC * NS)) // CH
    tail = (E // (NC * NS)) - n_chunks * CH

    idx = (idx0_v, idx1_v)
    sem = (sem0, sem1)

    def fire(i, b):
        pltpu.sync_copy(dst_hbm.at[pl.ds(ebase + i * CH, CH)], idx[b])
        pltpu.async_copy(ones_v, acc.at[idx[b]], sem[b], add=True)

    def drain(b):
        pltpu.make_async_copy(ones_v, acc.at[idx[b]], sem[b]).wait()

    for b in range(2):
        fire(b, b)

    def chunk(g, carry):
        for b in range(2):
            @pl.when(g * 2 + b + 2 < n_chunks)
            def _():
                drain(b)
                fire(g * 2 + b + 2, b)
        return carry

    lax.fori_loop(0, n_chunks // 2, chunk, 0)
    drain(0)
    drain(1)
    pltpu.sync_copy(dst_hbm.at[pl.ds(ebase + n_chunks * CH, tail)], tidx_v)
    pltpu.async_copy(ones_v.at[pl.ds(0, tail)], acc.at[tidx_v], tsem,
                     add=True).wait()

    plsc.subcore_barrier()
    base = c * N
    _slab_copy(s, lambda r0, n: acc.at[pl.ds(r0, n)],
               lambda r0, n: degp_hbm.at[pl.ds(base + r0, n)])


@jax.jit
def _deg_call(dst, zeros128, ones128):
    k = pl.kernel(
        _deg_body,
        out_type=jax.ShapeDtypeStruct((NC * N, 128), jnp.float32),
        mesh=plsc.VectorSubcoreMesh(**_MESH),
        scratch_types=[
            pltpu.VMEM((CH, 128), jnp.float32),
            pltpu.VMEM((CH,), jnp.int32),
            pltpu.VMEM((CH,), jnp.int32),
            pltpu.VMEM((16,), jnp.int32),
            pltpu.VMEM_SHARED((N, 128), jnp.float32),
            pltpu.SemaphoreType.DMA,
            pltpu.SemaphoreType.DMA,
            pltpu.SemaphoreType.DMA,
        ],
    )
    return k(dst, zeros128, ones128)


NBUF = 3


def _agg_body(split_cores, h_hbm, src_hbm, dst_hbm, zrows_hbm, agg_hbm,
              *scr):
    fidx = (scr[0:NBUF], scr[NBUF:2 * NBUF])
    didx = (scr[2 * NBUF:3 * NBUF], scr[3 * NBUF:4 * NBUF])
    rows = scr[4 * NBUF:5 * NBUF]
    tfidx_v, tdidx_v, acc = scr[5 * NBUF:5 * NBUF + 3]
    gsem = scr[5 * NBUF + 3:5 * NBUF + 3 + NBUF]
    ssem = scr[5 * NBUF + 3 + NBUF:5 * NBUF + 3 + 2 * NBUF]
    tsem = scr[-1]

    c = lax.axis_index("c")
    s = lax.axis_index("s")
    _slab_copy(s, lambda r0, n: zrows_hbm.at[pl.ds(0, n)],
               lambda r0, n: acc.at[pl.ds(r0, n)])
    plsc.subcore_barrier()

    if split_cores:
        e_per_tile = E // (NC * NS)
        ebase = (c * NS + s) * e_per_tile
        hoff = None
    else:
        e_per_tile = E // NS
        ebase = s * e_per_tile
        hoff = c * N
    n_chunks = e_per_tile // CH
    tail = e_per_tile - n_chunks * CH
    n_groups = n_chunks // NBUF
    leftover = n_chunks - n_groups * NBUF

    def load_idx(i, p, b):
        off = ebase + i * CH
        pltpu.sync_copy(src_hbm.at[pl.ds(off, CH)], fidx[p][b])
        pltpu.sync_copy(dst_hbm.at[pl.ds(off, CH)], didx[p][b])
        if hoff is not None:
            for j in range(CH // 16):
                fidx[p][b][pl.ds(j * 16, 16)] = (
                    fidx[p][b][pl.ds(j * 16, 16)] + hoff)

    def fire_gather(p, b):
        pltpu.async_copy(h_hbm.at[fidx[p][b]], rows[b], gsem[b])

    def wait_gather(p, b):
        pltpu.make_async_copy(h_hbm.at[fidx[p][b]], rows[b], gsem[b]).wait()

    def fire_scatter(p, b):
        pltpu.async_copy(rows[b], acc.at[didx[p][b]], ssem[b], add=True)

    def wait_scatter(p, b):
        pltpu.make_async_copy(rows[b], acc.at[didx[p][b]], ssem[b]).wait()

    for b in range(NBUF):
        load_idx(b, 0, b)
        fire_gather(0, b)

    def group2(g, carry):
        for b in range(NBUF):
            wait_gather(0, b)
            fire_scatter(0, b)

        @pl.when(2 * g + 1 < n_groups)
        def _():
            for b in range(NBUF):
                load_idx((2 * g + 1) * NBUF + b, 1, b)
            for b in range(NBUF):
                wait_scatter(0, b)
                fire_gather(1, b)
            for b in range(NBUF):
                wait_gather(1, b)
                fire_scatter(1, b)

            @pl.when(2 * g + 2 < n_groups)
            def _():
                for b in range(NBUF):
                    load_idx((2 * g + 2) * NBUF + b, 0, b)
                for b in range(NBUF):
                    wait_scatter(1, b)
                    fire_gather(0, b)

        return carry

    lax.fori_loop(0, (n_groups + 1) // 2, group2, 0)
    last_p = (n_groups - 1) % 2
    for b in range(NBUF):
        wait_scatter(last_p, b)

    for k in range(leftover):
        i = n_groups * NBUF + k
        load_idx(i, 0, 0)
        fire_gather(0, 0)
        wait_gather(0, 0)
        fire_scatter(0, 0)
        wait_scatter(0, 0)

    toff = ebase + n_chunks * CH
    pltpu.sync_copy(src_hbm.at[pl.ds(toff, tail)], tfidx_v)
    pltpu.sync_copy(dst_hbm.at[pl.ds(toff, tail)], tdidx_v)
    if hoff is not None:
        for j in range(tail // 16):
            tfidx_v[pl.ds(j * 16, 16)] = tfidx_v[pl.ds(j * 16, 16)] + hoff
    trows = rows[0].at[pl.ds(0, tail)]
    pltpu.async_copy(h_hbm.at[tfidx_v], trows, tsem).wait()
    pltpu.sync_copy(trows, acc.at[tdidx_v], add=True)

    plsc.subcore_barrier()
    base = c * N
    _slab_copy(s, lambda r0, n: acc.at[pl.ds(r0, n)],
               lambda r0, n: agg_hbm.at[pl.ds(base + r0, n)])


def _make_agg_call(split_cores, table_rows):
    e_per_tile = E // (NC * NS) if split_cores else E // NS
    tail = e_per_tile % CH

    @jax.jit
    def call(h_flat, src, dst, zrows):
        k = pl.kernel(
            functools.partial(_agg_body, split_cores),
            out_type=jax.ShapeDtypeStruct((NC * N, 128), jnp.float32),
            mesh=plsc.VectorSubcoreMesh(**_MESH),
            scratch_types=(
                [pltpu.VMEM((CH,), jnp.int32)] * (4 * NBUF)
                + [pltpu.VMEM((CH, 128), jnp.float32)] * NBUF
                + [
                    pltpu.VMEM((tail,), jnp.int32),
                    pltpu.VMEM((tail,), jnp.int32),
                    pltpu.VMEM_SHARED((N, 128), jnp.float32),
                ]
                + [pltpu.SemaphoreType.DMA] * (2 * NBUF + 1)
            ),
        )
        return k(h_flat, src, dst, zrows)

    return call


_agg_call_l1 = _make_agg_call(False, NC * N)
_agg_call_l2 = _make_agg_call(True, N)


BLK = 400


def _dinv_block(degp_blk):
    deg = degp_blk[0, :, 0:1] + degp_blk[1, :, 0:1]
    safe = jnp.where(deg > 0, deg, 1.0)
    return jnp.where(deg > 0, lax.rsqrt(safe), 0.0)


def _tc1a_body(x_ref, w1_ref, h1_ref):
    h = jnp.dot(x_ref[...], w1_ref[...], preferred_element_type=jnp.float32)
    h1_ref[0] = h[:, :128]
    h1_ref[1] = h[:, 128:]


@jax.jit
def _tc1a(x, w1p):
    return pl.pallas_call(
        _tc1a_body,
        grid=(N // BLK,),
        in_specs=[
            pl.BlockSpec((BLK, 128), lambda i: (i, 0)),
            pl.BlockSpec((128, 256), lambda i: (0, 0)),
        ],
        out_specs=pl.BlockSpec((2, BLK, 128), lambda i: (0, i, 0)),
        out_shape=jax.ShapeDtypeStruct((2, N, 128), jnp.float32),
    )(x, w1p)


def _tc1b_body(hraw_ref, degp_ref, h1_ref):
    dinv = _dinv_block(degp_ref[...])
    h1_ref[0] = hraw_ref[0] * dinv
    h1_ref[1] = hraw_ref[1] * dinv


@jax.jit
def _tc1b(hraw, degp3):
    return pl.pallas_call(
        _tc1b_body,
        grid=(N // BLK,),
        in_specs=[
            pl.BlockSpec((2, BLK, 128), lambda i: (0, i, 0)),
            pl.BlockSpec((2, BLK, 16), lambda i: (0, i, 0)),
        ],
        out_specs=pl.BlockSpec((2, BLK, 128), lambda i: (0, i, 0)),
        out_shape=jax.ShapeDtypeStruct((2, N, 128), jnp.float32),
    )(hraw, degp3)


def _tc2_body(agg_ref, b1e_ref, b1o_ref, degp_ref, w2_ref, h2_ref):
    dinv = _dinv_block(degp_ref[...])
    a0 = agg_ref[0] * dinv + b1e_ref[...]
    a1 = agg_ref[1] * dinv + b1o_ref[...]
    t = jnp.maximum(jnp.maximum(a0, 0.0), jnp.maximum(a1, 0.0))
    h2_ref[...] = jnp.dot(t, w2_ref[...],
                          preferred_element_type=jnp.float32) * dinv


@jax.jit
def _tc2(agg1, b1e, b1o, degp3, w2p):
    return pl.pallas_call(
        _tc2_body,
        grid=(N // BLK,),
        in_specs=[
            pl.BlockSpec((2, BLK, 128), lambda i: (0, i, 0)),
            pl.BlockSpec((1, 128), lambda i: (0, 0)),
            pl.BlockSpec((1, 128), lambda i: (0, 0)),
            pl.BlockSpec((2, BLK, 16), lambda i: (0, i, 0)),
            pl.BlockSpec((128, 128), lambda i: (0, 0)),
        ],
        out_specs=pl.BlockSpec((BLK, 128), lambda i: (i, 0)),
        out_shape=jax.ShapeDtypeStruct((N, 128), jnp.float32),
    )(agg1, b1e, b1o, degp3, w2p)


def _tc3_body(agg_ref, b2e_ref, b2o_ref, degp_ref, fc1w_ref, fc1b_ref,
              fc2w_ref, fc2b_ref, out_ref):
    dinv = _dinv_block(degp_ref[...])
    g = (agg_ref[0] + agg_ref[1]) * dinv
    a0 = g[:, :64] + b2e_ref[...]
    a1 = g[:, 64:] + b2o_ref[...]
    t = jnp.maximum(jnp.maximum(a0, 0.0), jnp.maximum(a1, 0.0))
    t = jnp.maximum(
        jnp.dot(t, fc1w_ref[...], preferred_element_type=jnp.float32)
        + fc1b_ref[...], 0.0)
    out_ref[...] = jnp.maximum(
        jnp.dot(t, fc2w_ref[...], preferred_element_type=jnp.float32)
        + fc2b_ref[...], 0.0)


@jax.jit
def _tc3(agg2, b2e, b2o, degp3, fc1_W, fc1_b, fc2_W, fc2_b):
    return pl.pallas_call(
        _tc3_body,
        grid=(N // BLK,),
        in_specs=[
            pl.BlockSpec((2, BLK, 128), lambda i: (0, i, 0)),
            pl.BlockSpec((1, 64), lambda i: (0, 0)),
            pl.BlockSpec((1, 64), lambda i: (0, 0)),
            pl.BlockSpec((2, BLK, 16), lambda i: (0, i, 0)),
            pl.BlockSpec((64, 64), lambda i: (0, 0)),
            pl.BlockSpec((1, 64), lambda i: (0, 0)),
            pl.BlockSpec((64, 40), lambda i: (0, 0)),
            pl.BlockSpec((1, 40), lambda i: (0, 0)),
        ],
        out_specs=pl.BlockSpec((BLK, 40), lambda i: (i, 0)),
        out_shape=jax.ShapeDtypeStruct((N, 40), jnp.float32),
    )(agg2, b2e, b2o, degp3, fc1_W, fc1_b, fc2_W, fc2_b)


def kernel(x, A, W1, b1, W2, b2, fc1_W, fc1_b, fc2_W, fc2_b):
    A = A.astype(jnp.int32)
    src = A[0]
    dst = A[1]
    W1p = jnp.concatenate([W1[:, 0::2], W1[:, 1::2]], axis=1)
    b1e = b1[0::2].reshape(1, 128)
    b1o = b1[1::2].reshape(1, 128)
    W2p = jnp.concatenate([W2[:, 0::2], W2[:, 1::2]], axis=1)
    b2e = b2[0::2].reshape(1, 64)
    b2o = b2[1::2].reshape(1, 64)

    ones128 = jnp.ones((CH, 128), jnp.float32)
    z128 = jnp.zeros((SLAB_LAST, 128), jnp.float32)

    degp3 = _deg_call(dst, z128, ones128).reshape(2, N, 128)[:, :, :16]
    hraw = _tc1a(x, W1p)
    h1 = _tc1b(hraw, degp3)
    agg1 = _agg_call_l1(h1.reshape(2 * N, 128), src, dst,
                        z128).reshape(2, N, 128)
    h2 = _tc2(agg1, b1e, b1o, degp3, W2p)
    agg2 = _agg_call_l2(h2, src, dst, z128).reshape(2, N, 128)
    return _tc3(agg2, b2e, b2o, degp3, fc1_W, fc1_b.reshape(1, 64),
                fc2_W, fc2_b.reshape(1, 40))


# speedup vs baseline: 1.2371x; 1.2371x over previous
import functools

import jax
import jax.numpy as jnp
from jax import lax
from jax.experimental import pallas as pl
from jax.experimental.pallas import tpu as pltpu
from jax.experimental.pallas import tpu_sc as plsc

N = 10000
E = 320000
NC = 2
NS = 16
SLAB = 624
SLAB_LAST = N - SLAB * (NS - 1)
CH = 128

_MESH = dict(core_axis_name="c", subcore_axis_name="s", num_cores=NC,
             num_subcores=NS)


def _slab_copy(s, mk_src, mk_dst):
    @pl.when(s < NS - 1)
    def _():
        pltpu.sync_copy(mk_src(s * SLAB, SLAB), mk_dst(s * SLAB, SLAB))

    @pl.when(s == NS - 1)
    def _():
        pltpu.sync_copy(mk_src(SLAB * (NS - 1), SLAB_LAST),
                        mk_dst(SLAB * (NS - 1), SLAB_LAST))


def _deg_body(dst_hbm, zeros_hbm, ones_hbm, degp_hbm, ones_v, idx0_v, idx1_v,
              tidx_v, acc, sem0, sem1, tsem):
    c = lax.axis_index("c")
    s = lax.axis_index("s")
    _slab_copy(s, lambda r0, n: zeros_hbm.at[pl.ds(0, n)],
               lambda r0, n: acc.at[pl.ds(r0, n)])
    pltpu.sync_copy(ones_hbm, ones_v)
    plsc.subcore_barrier()

    ebase = (c * NS + s) * (E // (NC * NS))
    n_chunks = (E // (NC * NS)) // CH
    tail = (E // (NC * NS)) - n_chunks * CH

    idx = (idx0_v, idx1_v)
    sem = (sem0, sem1)

    def fire(i, b):
        pltpu.sync_copy(dst_hbm.at[pl.ds(ebase + i * CH, CH)], idx[b])
        pltpu.async_copy(ones_v, acc.at[idx[b]], sem[b], add=True)

    def drain(b):
        pltpu.make_async_copy(ones_v, acc.at[idx[b]], sem[b]).wait()

    for b in range(2):
        fire(b, b)

    def chunk(g, carry):
        for b in range(2):
            @pl.when(g * 2 + b + 2 < n_chunks)
            def _():
                drain(b)
                fire(g * 2 + b + 2, b)
        return carry

    lax.fori_loop(0, n_chunks // 2, chunk, 0)
    drain(0)
    drain(1)
    pltpu.sync_copy(dst_hbm.at[pl.ds(ebase + n_chunks * CH, tail)], tidx_v)
    pltpu.async_copy(ones_v.at[pl.ds(0, tail)], acc.at[tidx_v], tsem,
                     add=True).wait()

    plsc.subcore_barrier()
    base = c * N
    _slab_copy(s, lambda r0, n: acc.at[pl.ds(r0, n)],
               lambda r0, n: degp_hbm.at[pl.ds(base + r0, n)])


@jax.jit
def _deg_call(dst, zeros128, ones128):
    k = pl.kernel(
        _deg_body,
        out_type=jax.ShapeDtypeStruct((NC * N, 128), jnp.float32),
        mesh=plsc.VectorSubcoreMesh(**_MESH),
        scratch_types=[
            pltpu.VMEM((CH, 128), jnp.float32),
            pltpu.VMEM((CH,), jnp.int32),
            pltpu.VMEM((CH,), jnp.int32),
            pltpu.VMEM((16,), jnp.int32),
            pltpu.VMEM_SHARED((N, 128), jnp.float32),
            pltpu.SemaphoreType.DMA,
            pltpu.SemaphoreType.DMA,
            pltpu.SemaphoreType.DMA,
        ],
    )
    return k(dst, zeros128, ones128)


NBUF = 3


def _agg_body(split_cores, h_hbm, src_hbm, dst_hbm, zrows_hbm, agg_hbm,
              *scr):
    fidx = scr[0:NBUF]
    didx = scr[NBUF:2 * NBUF]
    rows = scr[2 * NBUF:3 * NBUF]
    tfidx_v, tdidx_v, acc = scr[3 * NBUF:3 * NBUF + 3]
    gsem = scr[3 * NBUF + 3:3 * NBUF + 3 + NBUF]
    ssem = scr[3 * NBUF + 3 + NBUF:3 * NBUF + 3 + 2 * NBUF]
    tsem = scr[-1]

    c = lax.axis_index("c")
    s = lax.axis_index("s")
    _slab_copy(s, lambda r0, n: zrows_hbm.at[pl.ds(0, n)],
               lambda r0, n: acc.at[pl.ds(r0, n)])
    plsc.subcore_barrier()

    if split_cores:
        e_per_tile = E // (NC * NS)
        ebase = (c * NS + s) * e_per_tile
        hoff = None
    else:
        e_per_tile = E // NS
        ebase = s * e_per_tile
        hoff = c * N
    n_chunks = e_per_tile // CH
    tail = e_per_tile - n_chunks * CH
    n_groups = n_chunks // NBUF
    leftover = n_chunks - n_groups * NBUF

    def fire_gather(i, b):
        off = ebase + i * CH
        pltpu.sync_copy(src_hbm.at[pl.ds(off, CH)], fidx[b])
        pltpu.sync_copy(dst_hbm.at[pl.ds(off, CH)], didx[b])
        if hoff is not None:
            for j in range(CH // 16):
                fidx[b][pl.ds(j * 16, 16)] = (
                    fidx[b][pl.ds(j * 16, 16)] + hoff)
        pltpu.async_copy(h_hbm.at[fidx[b]], rows[b], gsem[b])

    def wait_gather(b):
        pltpu.make_async_copy(h_hbm.at[fidx[b]], rows[b], gsem[b]).wait()

    def fire_scatter(b):
        pltpu.async_copy(rows[b], acc.at[didx[b]], ssem[b], add=True)

    def wait_scatter(b):
        pltpu.make_async_copy(rows[b], acc.at[didx[b]], ssem[b]).wait()

    for b in range(NBUF):
        fire_gather(b, b)

    def group(g, carry):
        for b in range(NBUF):
            wait_gather(b)
            fire_scatter(b)

        @pl.when(g + 1 < n_groups)
        def _():
            for b in range(NBUF):
                wait_scatter(b)
                fire_gather((g + 1) * NBUF + b, b)

        return carry

    lax.fori_loop(0, n_groups, group, 0)
    for b in range(NBUF):
        wait_scatter(b)

    for k in range(leftover):
        i = n_groups * NBUF + k
        fire_gather(i, 0)
        wait_gather(0)
        fire_scatter(0)
        wait_scatter(0)

    toff = ebase + n_chunks * CH
    pltpu.sync_copy(src_hbm.at[pl.ds(toff, tail)], tfidx_v)
    pltpu.sync_copy(dst_hbm.at[pl.ds(toff, tail)], tdidx_v)
    if hoff is not None:
        for j in range(tail // 16):
            tfidx_v[pl.ds(j * 16, 16)] = tfidx_v[pl.ds(j * 16, 16)] + hoff
    trows = rows[0].at[pl.ds(0, tail)]
    pltpu.async_copy(h_hbm.at[tfidx_v], trows, tsem).wait()
    pltpu.sync_copy(trows, acc.at[tdidx_v], add=True)

    plsc.subcore_barrier()
    base = c * N
    _slab_copy(s, lambda r0, n: acc.at[pl.ds(r0, n)],
               lambda r0, n: agg_hbm.at[pl.ds(base + r0, n)])


def _make_agg_call(split_cores, table_rows):
    e_per_tile = E // (NC * NS) if split_cores else E // NS
    tail = e_per_tile % CH

    @jax.jit
    def call(h_flat, src, dst, zrows):
        k = pl.kernel(
            functools.partial(_agg_body, split_cores),
            out_type=jax.ShapeDtypeStruct((NC * N, 128), jnp.float32),
            mesh=plsc.VectorSubcoreMesh(**_MESH),
            scratch_types=(
                [pltpu.VMEM((CH,), jnp.int32)] * (2 * NBUF)
                + [pltpu.VMEM((CH, 128), jnp.float32)] * NBUF
                + [
                    pltpu.VMEM((tail,), jnp.int32),
                    pltpu.VMEM((tail,), jnp.int32),
                    pltpu.VMEM_SHARED((N, 128), jnp.float32),
                ]
                + [pltpu.SemaphoreType.DMA] * (2 * NBUF + 1)
            ),
        )
        return k(h_flat, src, dst, zrows)

    return call


_agg_call_l1 = _make_agg_call(False, NC * N)
_agg_call_l2 = _make_agg_call(True, N)


BLK = 400


def _dinv_block(degp_blk):
    deg = degp_blk[0, :, 0:1] + degp_blk[1, :, 0:1]
    safe = jnp.where(deg > 0, deg, 1.0)
    return jnp.where(deg > 0, lax.rsqrt(safe), 0.0)


def _tc1a_body(x_ref, w1_ref, h1_ref):
    h = jnp.dot(x_ref[...], w1_ref[...], preferred_element_type=jnp.float32)
    h1_ref[0] = h[:, :128]
    h1_ref[1] = h[:, 128:]


@jax.jit
def _tc1a(x, w1p):
    return pl.pallas_call(
        _tc1a_body,
        grid=(N // BLK,),
        in_specs=[
            pl.BlockSpec((BLK, 128), lambda i: (i, 0)),
            pl.BlockSpec((128, 256), lambda i: (0, 0)),
        ],
        out_specs=pl.BlockSpec((2, BLK, 128), lambda i: (0, i, 0)),
        out_shape=jax.ShapeDtypeStruct((2, N, 128), jnp.float32),
    )(x, w1p)


def _tc1b_body(hraw_ref, degp_ref, h1_ref):
    dinv = _dinv_block(degp_ref[...])
    h1_ref[0] = hraw_ref[0] * dinv
    h1_ref[1] = hraw_ref[1] * dinv


@jax.jit
def _tc1b(hraw, degp3):
    return pl.pallas_call(
        _tc1b_body,
        grid=(N // BLK,),
        in_specs=[
            pl.BlockSpec((2, BLK, 128), lambda i: (0, i, 0)),
            pl.BlockSpec((2, BLK, 16), lambda i: (0, i, 0)),
        ],
        out_specs=pl.BlockSpec((2, BLK, 128), lambda i: (0, i, 0)),
        out_shape=jax.ShapeDtypeStruct((2, N, 128), jnp.float32),
    )(hraw, degp3)


def _tc2_body(agg_ref, b1e_ref, b1o_ref, degp_ref, w2_ref, h2_ref):
    dinv = _dinv_block(degp_ref[...])
    a0 = agg_ref[0] * dinv + b1e_ref[...]
    a1 = agg_ref[1] * dinv + b1o_ref[...]
    t = jnp.maximum(jnp.maximum(a0, 0.0), jnp.maximum(a1, 0.0))
    h2_ref[...] = jnp.dot(t, w2_ref[...],
                          preferred_element_type=jnp.float32) * dinv


@jax.jit
def _tc2(agg1, b1e, b1o, degp3, w2p):
    return pl.pallas_call(
        _tc2_body,
        grid=(N // BLK,),
        in_specs=[
            pl.BlockSpec((2, BLK, 128), lambda i: (0, i, 0)),
            pl.BlockSpec((1, 128), lambda i: (0, 0)),
            pl.BlockSpec((1, 128), lambda i: (0, 0)),
            pl.BlockSpec((2, BLK, 16), lambda i: (0, i, 0)),
            pl.BlockSpec((128, 128), lambda i: (0, 0)),
        ],
        out_specs=pl.BlockSpec((BLK, 128), lambda i: (i, 0)),
        out_shape=jax.ShapeDtypeStruct((N, 128), jnp.float32),
    )(agg1, b1e, b1o, degp3, w2p)


def _tc3_body(agg_ref, b2e_ref, b2o_ref, degp_ref, fc1w_ref, fc1b_ref,
              fc2w_ref, fc2b_ref, out_ref):
    dinv = _dinv_block(degp_ref[...])
    g = (agg_ref[0] + agg_ref[1]) * dinv
    a0 = g[:, :64] + b2e_ref[...]
    a1 = g[:, 64:] + b2o_ref[...]
    t = jnp.maximum(jnp.maximum(a0, 0.0), jnp.maximum(a1, 0.0))
    t = jnp.maximum(
        jnp.dot(t, fc1w_ref[...], preferred_element_type=jnp.float32)
        + fc1b_ref[...], 0.0)
    out_ref[...] = jnp.maximum(
        jnp.dot(t, fc2w_ref[...], preferred_element_type=jnp.float32)
        + fc2b_ref[...], 0.0)


@jax.jit
def _tc3(agg2, b2e, b2o, degp3, fc1_W, fc1_b, fc2_W, fc2_b):
    return pl.pallas_call(
        _tc3_body,
        grid=(N // BLK,),
        in_specs=[
            pl.BlockSpec((2, BLK, 128), lambda i: (0, i, 0)),
            pl.BlockSpec((1, 64), lambda i: (0, 0)),
            pl.BlockSpec((1, 64), lambda i: (0, 0)),
            pl.BlockSpec((2, BLK, 16), lambda i: (0, i, 0)),
            pl.BlockSpec((64, 64), lambda i: (0, 0)),
            pl.BlockSpec((1, 64), lambda i: (0, 0)),
            pl.BlockSpec((64, 40), lambda i: (0, 0)),
            pl.BlockSpec((1, 40), lambda i: (0, 0)),
        ],
        out_specs=pl.BlockSpec((BLK, 40), lambda i: (i, 0)),
        out_shape=jax.ShapeDtypeStruct((N, 40), jnp.float32),
    )(agg2, b2e, b2o, degp3, fc1_W, fc1_b, fc2_W, fc2_b)


def kernel(x, A, W1, b1, W2, b2, fc1_W, fc1_b, fc2_W, fc2_b):
    A = A.astype(jnp.int32)
    src = A[0]
    dst = A[1]
    W1p = jnp.concatenate([W1[:, 0::2], W1[:, 1::2]], axis=1)
    b1e = b1[0::2].reshape(1, 128)
    b1o = b1[1::2].reshape(1, 128)
    W2p = jnp.concatenate([W2[:, 0::2], W2[:, 1::2]], axis=1)
    b2e = b2[0::2].reshape(1, 64)
    b2o = b2[1::2].reshape(1, 64)

    ones128 = jnp.ones((CH, 128), jnp.float32)
    z128 = jnp.zeros((SLAB_LAST, 128), jnp.float32)

    degp3 = _deg_call(dst, z128, ones128).reshape(2, N, 128)[:, :, :16]
    hraw = _tc1a(x, W1p)
    h1 = _tc1b(hraw, degp3)
    agg1 = _agg_call_l1(h1.reshape(2 * N, 128), src, dst,
                        z128).reshape(2, N, 128)
    h2 = _tc2(agg1, b1e, b1o, degp3, W2p)
    agg2 = _agg_call_l2(h2, src, dst, z128).reshape(2, N, 128)
    return _tc3(agg2, b2e, b2o, degp3, fc1_W, fc1_b.reshape(1, 64),
                fc2_W, fc2_b.reshape(1, 40))


# speedup vs baseline: 1.2408x; 1.0029x over previous
import functools

import jax
import jax.numpy as jnp
from jax import lax
from jax.experimental import pallas as pl
from jax.experimental.pallas import tpu as pltpu
from jax.experimental.pallas import tpu_sc as plsc

N = 10000
E = 320000
NC = 2
NS = 16
SLAB = 624
SLAB_LAST = N - SLAB * (NS - 1)
CH = 128

_MESH = dict(core_axis_name="c", subcore_axis_name="s", num_cores=NC,
             num_subcores=NS)


def _slab_copy(s, mk_src, mk_dst):
    @pl.when(s < NS - 1)
    def _():
        pltpu.sync_copy(mk_src(s * SLAB, SLAB), mk_dst(s * SLAB, SLAB))

    @pl.when(s == NS - 1)
    def _():
        pltpu.sync_copy(mk_src(SLAB * (NS - 1), SLAB_LAST),
                        mk_dst(SLAB * (NS - 1), SLAB_LAST))


def _deg_body(dst_hbm, zeros_hbm, ones_hbm, degp_hbm, ones_v, idx0_v, idx1_v,
              tidx_v, acc, sem0, sem1, tsem):
    c = lax.axis_index("c")
    s = lax.axis_index("s")
    _slab_copy(s, lambda r0, n: zeros_hbm.at[pl.ds(0, n)],
               lambda r0, n: acc.at[pl.ds(r0, n)])
    pltpu.sync_copy(ones_hbm, ones_v)
    plsc.subcore_barrier()

    ebase = (c * NS + s) * (E // (NC * NS))
    n_chunks = (E // (NC * NS)) // CH
    tail = (E // (NC * NS)) - n_chunks * CH

    idx = (idx0_v, idx1_v)
    sem = (sem0, sem1)

    def fire(i, b):
        pltpu.sync_copy(dst_hbm.at[pl.ds(ebase + i * CH, CH)], idx[b])
        pltpu.async_copy(ones_v, acc.at[idx[b]], sem[b], add=True)

    def drain(b):
        pltpu.make_async_copy(ones_v, acc.at[idx[b]], sem[b]).wait()

    for b in range(2):
        fire(b, b)

    def chunk(g, carry):
        for b in range(2):
            @pl.when(g * 2 + b + 2 < n_chunks)
            def _():
                drain(b)
                fire(g * 2 + b + 2, b)
        return carry

    lax.fori_loop(0, n_chunks // 2, chunk, 0)
    drain(0)
    drain(1)
    pltpu.sync_copy(dst_hbm.at[pl.ds(ebase + n_chunks * CH, tail)], tidx_v)
    pltpu.async_copy(ones_v.at[pl.ds(0, tail)], acc.at[tidx_v], tsem,
                     add=True).wait()

    plsc.subcore_barrier()
    base = c * N
    _slab_copy(s, lambda r0, n: acc.at[pl.ds(r0, n)],
               lambda r0, n: degp_hbm.at[pl.ds(base + r0, n)])


@jax.jit
def _deg_call(dst, zeros128, ones128):
    k = pl.kernel(
        _deg_body,
        out_type=jax.ShapeDtypeStruct((NC * N, 128), jnp.float32),
        mesh=plsc.VectorSubcoreMesh(**_MESH),
        scratch_types=[
            pltpu.VMEM((CH, 128), jnp.float32),
            pltpu.VMEM((CH,), jnp.int32),
            pltpu.VMEM((CH,), jnp.int32),
            pltpu.VMEM((16,), jnp.int32),
            pltpu.VMEM_SHARED((N, 128), jnp.float32),
            pltpu.SemaphoreType.DMA,
            pltpu.SemaphoreType.DMA,
            pltpu.SemaphoreType.DMA,
        ],
    )
    return k(dst, zeros128, ones128)


NBUF = 3


def _agg_body(split_cores, h_hbm, src_hbm, dst_hbm, zrows_hbm, agg_hbm,
              *scr):
    fidx = scr[0:NBUF]
    didx = scr[NBUF:2 * NBUF]
    rows = scr[2 * NBUF:3 * NBUF]
    tfidx_v, tdidx_v, acc = scr[3 * NBUF:3 * NBUF + 3]
    gsem = scr[3 * NBUF + 3:3 * NBUF + 3 + NBUF]
    ssem = scr[3 * NBUF + 3 + NBUF:3 * NBUF + 3 + 2 * NBUF]
    tsem = scr[-1]

    c = lax.axis_index("c")
    s = lax.axis_index("s")

    if split_cores:
        e_per_tile = E // (NC * NS)
        ebase = (c * NS + s) * e_per_tile
        hoff = None
    else:
        e_per_tile = E // NS
        ebase = s * e_per_tile
        hoff = c * N
    n_chunks = e_per_tile // CH
    tail = e_per_tile - n_chunks * CH
    n_groups = n_chunks // NBUF
    leftover = n_chunks - n_groups * NBUF

    def fire_gather(i, b):
        off = ebase + i * CH
        pltpu.sync_copy(src_hbm.at[pl.ds(off, CH)], fidx[b])
        pltpu.sync_copy(dst_hbm.at[pl.ds(off, CH)], didx[b])
        if hoff is not None:
            for j in range(CH // 16):
                fidx[b][pl.ds(j * 16, 16)] = (
                    fidx[b][pl.ds(j * 16, 16)] + hoff)
        pltpu.async_copy(h_hbm.at[fidx[b]], rows[b], gsem[b])

    def wait_gather(b):
        pltpu.make_async_copy(h_hbm.at[fidx[b]], rows[b], gsem[b]).wait()

    def fire_scatter(b):
        pltpu.async_copy(rows[b], acc.at[didx[b]], ssem[b], add=True)

    def wait_scatter(b):
        pltpu.make_async_copy(rows[b], acc.at[didx[b]], ssem[b]).wait()

    for b in range(NBUF):
        fire_gather(b, b)
    _slab_copy(s, lambda r0, n: zrows_hbm.at[pl.ds(0, n)],
               lambda r0, n: acc.at[pl.ds(r0, n)])
    plsc.subcore_barrier()

    def group(g, carry):
        for b in range(NBUF):
            wait_gather(b)
            fire_scatter(b)

        @pl.when(g + 1 < n_groups)
        def _():
            for b in range(NBUF):
                wait_scatter(b)
                fire_gather((g + 1) * NBUF + b, b)

        return carry

    lax.fori_loop(0, n_groups, group, 0)
    for b in range(NBUF):
        wait_scatter(b)

    for k in range(leftover):
        i = n_groups * NBUF + k
        fire_gather(i, 0)
        wait_gather(0)
        fire_scatter(0)
        wait_scatter(0)

    toff = ebase + n_chunks * CH
    pltpu.sync_copy(src_hbm.at[pl.ds(toff, tail)], tfidx_v)
    pltpu.sync_copy(dst_hbm.at[pl.ds(toff, tail)], tdidx_v)
    if hoff is not None:
        for j in range(tail // 16):
            tfidx_v[pl.ds(j * 16, 16)] = tfidx_v[pl.ds(j * 16, 16)] + hoff
    trows = rows[0].at[pl.ds(0, tail)]
    pltpu.async_copy(h_hbm.at[tfidx_v], trows, tsem).wait()
    pltpu.sync_copy(trows, acc.at[tdidx_v], add=True)

    plsc.subcore_barrier()
    base = c * N
    _slab_copy(s, lambda r0, n: acc.at[pl.ds(r0, n)],
               lambda r0, n: agg_hbm.at[pl.ds(base + r0, n)])


def _make_agg_call(split_cores, table_rows):
    e_per_tile = E // (NC * NS) if split_cores else E // NS
    tail = e_per_tile % CH

    @jax.jit
    def call(h_flat, src, dst, zrows):
        k = pl.kernel(
            functools.partial(_agg_body, split_cores),
            out_type=jax.ShapeDtypeStruct((NC * N, 128), jnp.float32),
            mesh=plsc.VectorSubcoreMesh(**_MESH),
            scratch_types=(
                [pltpu.VMEM((CH,), jnp.int32)] * (2 * NBUF)
                + [pltpu.VMEM((CH, 128), jnp.float32)] * NBUF
                + [
                    pltpu.VMEM((tail,), jnp.int32),
                    pltpu.VMEM((tail,), jnp.int32),
                    pltpu.VMEM_SHARED((N, 128), jnp.float32),
                ]
                + [pltpu.SemaphoreType.DMA] * (2 * NBUF + 1)
            ),
        )
        return k(h_flat, src, dst, zrows)

    return call


_agg_call_l1 = _make_agg_call(False, NC * N)
_agg_call_l2 = _make_agg_call(True, N)


BLK = 400


def _dinv_block(degp_blk):
    deg = degp_blk[0, :, 0:1] + degp_blk[1, :, 0:1]
    safe = jnp.where(deg > 0, deg, 1.0)
    return jnp.where(deg > 0, lax.rsqrt(safe), 0.0)


def _tc1_body(x_ref, w1_ref, degp_ref, h1_ref):
    dinv = _dinv_block(degp_ref[...])
    h = jnp.dot(x_ref[...], w1_ref[...],
                preferred_element_type=jnp.float32) * dinv
    h1_ref[0] = h[:, :128]
    h1_ref[1] = h[:, 128:]


@jax.jit
def _tc1(x, w1p, degp3):
    return pl.pallas_call(
        _tc1_body,
        grid=(N // BLK,),
        in_specs=[
            pl.BlockSpec((BLK, 128), lambda i: (i, 0)),
            pl.BlockSpec((128, 256), lambda i: (0, 0)),
            pl.BlockSpec((2, BLK, 16), lambda i: (0, i, 0)),
        ],
        out_specs=pl.BlockSpec((2, BLK, 128), lambda i: (0, i, 0)),
        out_shape=jax.ShapeDtypeStruct((2, N, 128), jnp.float32),
    )(x, w1p, degp3)


def _tc2_body(agg_ref, b1e_ref, b1o_ref, degp_ref, w2_ref, h2_ref):
    dinv = _dinv_block(degp_ref[...])
    a0 = agg_ref[0] * dinv + b1e_ref[...]
    a1 = agg_ref[1] * dinv + b1o_ref[...]
    t = jnp.maximum(jnp.maximum(a0, 0.0), jnp.maximum(a1, 0.0))
    h2_ref[...] = jnp.dot(t, w2_ref[...],
                          preferred_element_type=jnp.float32) * dinv


@jax.jit
def _tc2(agg1, b1e, b1o, degp3, w2p):
    return pl.pallas_call(
        _tc2_body,
        grid=(N // BLK,),
        in_specs=[
            pl.BlockSpec((2, BLK, 128), lambda i: (0, i, 0)),
            pl.BlockSpec((1, 128), lambda i: (0, 0)),
            pl.BlockSpec((1, 128), lambda i: (0, 0)),
            pl.BlockSpec((2, BLK, 16), lambda i: (0, i, 0)),
            pl.BlockSpec((128, 128), lambda i: (0, 0)),
        ],
        out_specs=pl.BlockSpec((BLK, 128), lambda i: (i, 0)),
        out_shape=jax.ShapeDtypeStruct((N, 128), jnp.float32),
    )(agg1, b1e, b1o, degp3, w2p)


def _tc3_body(agg_ref, b2e_ref, b2o_ref, degp_ref, fc1w_ref, fc1b_ref,
              fc2w_ref, fc2b_ref, out_ref):
    dinv = _dinv_block(degp_ref[...])
    g = (agg_ref[0] + agg_ref[1]) * dinv
    a0 = g[:, :64] + b2e_ref[...]
    a1 = g[:, 64:] + b2o_ref[...]
    t = jnp.maximum(jnp.maximum(a0, 0.0), jnp.maximum(a1, 0.0))
    t = jnp.maximum(
        jnp.dot(t, fc1w_ref[...], preferred_element_type=jnp.float32)
        + fc1b_ref[...], 0.0)
    out_ref[...] = jnp.maximum(
        jnp.dot(t, fc2w_ref[...], preferred_element_type=jnp.float32)
        + fc2b_ref[...], 0.0)


@jax.jit
def _tc3(agg2, b2e, b2o, degp3, fc1_W, fc1_b, fc2_W, fc2_b):
    return pl.pallas_call(
        _tc3_body,
        grid=(N // BLK,),
        in_specs=[
            pl.BlockSpec((2, BLK, 128), lambda i: (0, i, 0)),
            pl.BlockSpec((1, 64), lambda i: (0, 0)),
            pl.BlockSpec((1, 64), lambda i: (0, 0)),
            pl.BlockSpec((2, BLK, 16), lambda i: (0, i, 0)),
            pl.BlockSpec((64, 64), lambda i: (0, 0)),
            pl.BlockSpec((1, 64), lambda i: (0, 0)),
            pl.BlockSpec((64, 40), lambda i: (0, 0)),
            pl.BlockSpec((1, 40), lambda i: (0, 0)),
        ],
        out_specs=pl.BlockSpec((BLK, 40), lambda i: (i, 0)),
        out_shape=jax.ShapeDtypeStruct((N, 40), jnp.float32),
    )(agg2, b2e, b2o, degp3, fc1_W, fc1_b, fc2_W, fc2_b)


def kernel(x, A, W1, b1, W2, b2, fc1_W, fc1_b, fc2_W, fc2_b):
    A = A.astype(jnp.int32)
    src = A[0]
    dst = A[1]
    W1p = jnp.concatenate([W1[:, 0::2], W1[:, 1::2]], axis=1)
    b1e = b1[0::2].reshape(1, 128)
    b1o = b1[1::2].reshape(1, 128)
    W2p = jnp.concatenate([W2[:, 0::2], W2[:, 1::2]], axis=1)
    b2e = b2[0::2].reshape(1, 64)
    b2o = b2[1::2].reshape(1, 64)

    ones128 = jnp.ones((CH, 128), jnp.float32)
    z128 = jnp.zeros((SLAB_LAST, 128), jnp.float32)

    degp3 = _deg_call(dst, z128, ones128).reshape(2, N, 128)[:, :, :16]
    h1 = _tc1(x, W1p, degp3)
    agg1 = _agg_call_l1(h1.reshape(2 * N, 128), src, dst,
                        z128).reshape(2, N, 128)
    h2 = _tc2(agg1, b1e, b1o, degp3, W2p)
    agg2 = _agg_call_l2(h2, src, dst, z128).reshape(2, N, 128)
    return _tc3(agg2, b2e, b2o, degp3, fc1_W, fc1_b.reshape(1, 64),
                fc2_W, fc2_b.reshape(1, 40))
